# baseline (device time: 1425742 ns/iter reference)
import jax
import jax.numpy as jnp
from jax import lax
from jax.experimental import pallas as pl
from jax.experimental.pallas import tpu as pltpu

N_DEV = 4
E_PER_SHARD = 4
N_TOK = 4096
D = 1024
H = 2048
CAP = 1280
BM = 512


def _partial_body(e_ref, x_ref, w_ref, out_ref):
    j = pl.program_id(1)
    my = lax.axis_index("i")
    m = e_ref[:, :] == my * E_PER_SHARD + j
    xm = jnp.where(m, x_ref[:, :], 0.0)
    contrib = jnp.dot(xm, w_ref[0], preferred_element_type=jnp.float32)

    @pl.when(j == 0)
    def _():
        out_ref[:, :] = contrib

    @pl.when(j != 0)
    def _():
        out_ref[:, :] += contrib


def _compute_partial(route_idx, x, expert_W):
    return pl.pallas_call(
        _partial_body,
        grid=(N_TOK // BM, E_PER_SHARD),
        in_specs=[
            pl.BlockSpec((BM, 1), lambda c, j: (c, 0)),
            pl.BlockSpec((BM, D), lambda c, j: (c, 0)),
            pl.BlockSpec((1, D, H), lambda c, j: (j, 0, 0)),
        ],
        out_specs=pl.BlockSpec((BM, H), lambda c, j: (c, 0)),
        out_shape=jax.ShapeDtypeStruct((N_TOK, H), jnp.float32),
    )(route_idx, x, expert_W)


def _scatter_body(
    cnt_ref, rows_ref, part_ref, out_ref, dump_ref,
    send_sem, recv_sem, copy_sem, bar_sem,
):
    my = lax.axis_index("i")
    cnt = cnt_ref[0]

    bulk = pltpu.make_async_copy(part_ref, out_ref, copy_sem)
    bulk.start()
    bulk.wait()

    for k in range(1, N_DEV):
        pl.semaphore_signal(
            bar_sem, inc=1,
            device_id=(lax.rem(my + k, N_DEV),),
            device_id_type=pl.DeviceIdType.MESH,
        )
    pl.semaphore_wait(bar_sem, N_DEV - 1)

    def send_row(r, carry):
        row = rows_ref[r]

        @pl.when(r < cnt)
        def _():
            for k in range(1, N_DEV):
                pltpu.make_async_remote_copy(
                    src_ref=part_ref.at[pl.ds(row, 1)],
                    dst_ref=out_ref.at[pl.ds(row, 1)],
                    send_sem=send_sem,
                    recv_sem=recv_sem,
                    device_id=(lax.rem(my + k, N_DEV),),
                    device_id_type=pl.DeviceIdType.MESH,
                ).start()

        @pl.when(r >= cnt)
        def _():
            for k in range(1, N_DEV):
                pltpu.make_async_remote_copy(
                    src_ref=part_ref.at[pl.ds(0, 1)],
                    dst_ref=dump_ref.at[pl.ds(0, 1)],
                    send_sem=send_sem,
                    recv_sem=recv_sem,
                    device_id=(lax.rem(my + k, N_DEV),),
                    device_id_type=pl.DeviceIdType.MESH,
                ).start()

        return carry

    lax.fori_loop(0, CAP, send_row, 0)

    total = (N_DEV - 1) * CAP
    dummy = pltpu.make_async_remote_copy(
        src_ref=out_ref.at[pl.ds(0, total)],
        dst_ref=out_ref.at[pl.ds(0, total)],
        send_sem=send_sem,
        recv_sem=recv_sem,
        device_id=(my,),
        device_id_type=pl.DeviceIdType.MESH,
    )
    dummy.wait_send()
    dummy.wait_recv()


def _scatter_combine(cnt, my_rows, partial):
    return pl.pallas_call(
        _scatter_body,
        in_specs=[
            pl.BlockSpec(memory_space=pltpu.MemorySpace.SMEM),
            pl.BlockSpec(memory_space=pltpu.MemorySpace.SMEM),
            pl.BlockSpec(memory_space=pltpu.MemorySpace.HBM),
        ],
        out_specs=pl.BlockSpec(memory_space=pltpu.MemorySpace.HBM),
        out_shape=jax.ShapeDtypeStruct((N_TOK, H), jnp.float32),
        scratch_shapes=[
            pltpu.VMEM((8, H), jnp.float32),
            pltpu.SemaphoreType.DMA,
            pltpu.SemaphoreType.DMA,
            pltpu.SemaphoreType.DMA,
            pltpu.SemaphoreType.REGULAR,
        ],
    )(cnt, my_rows, partial)


def kernel(x, router_W, route_idx, expert_W):
    del router_W
    e = route_idx[:, 0].astype(jnp.int32)
    my = lax.axis_index("i")

    mine = (e // E_PER_SHARD) == my
    cnt = jnp.sum(mine.astype(jnp.int32)).reshape(1)
    my_rows = jnp.argsort(jnp.where(mine, 0, 1), stable=True)[:CAP]
    my_rows = my_rows.astype(jnp.int32)

    partial = _compute_partial(route_idx.astype(jnp.int32), x, expert_W)
    return _scatter_combine(cnt, my_rows, partial)


# device time: 366489 ns/iter; 3.8903x vs baseline; 3.8903x over previous
import jax
import jax.numpy as jnp
from jax import lax
from jax.experimental import pallas as pl
from jax.experimental.pallas import tpu as pltpu

N_DEV = 4
E_PER_SHARD = 4
N_TOK = 4096
D = 1024
H = 2048
CAP = 1280

BK_G = 1024
BM_C = 256
BM_S = 512
BK_S = 640


def _gather_body(rows_ref, x_ref, o_ref):
    k = pl.program_id(0)
    t = k * BK_G + lax.broadcasted_iota(jnp.int32, (1, BK_G), 1)
    p = (rows_ref[:, :] == t).astype(jnp.bfloat16)
    xb = x_ref[:, :].astype(jnp.bfloat16)
    contrib = jnp.dot(p, xb, preferred_element_type=jnp.float32)

    @pl.when(k == 0)
    def _():
        o_ref[:, :] = contrib

    @pl.when(k != 0)
    def _():
        o_ref[:, :] += contrib


def _gather_x(rows2, x):
    return pl.pallas_call(
        _gather_body,
        grid=(N_TOK // BK_G,),
        in_specs=[
            pl.BlockSpec((CAP, 1), lambda k: (0, 0)),
            pl.BlockSpec((BK_G, D), lambda k: (k, 0)),
        ],
        out_specs=pl.BlockSpec((CAP, D), lambda k: (0, 0)),
        out_shape=jax.ShapeDtypeStruct((CAP, D), jnp.float32),
    )(rows2, x)


def _compact_body(e_ref, x_ref, w_ref, o_ref):
    j = pl.program_id(1)
    m = e_ref[:, :] == j
    xm = jnp.where(m, x_ref[:, :], 0.0)
    contrib = jnp.dot(xm, w_ref[0], preferred_element_type=jnp.float32)

    @pl.when(j == 0)
    def _():
        o_ref[:, :] = contrib.astype(jnp.bfloat16)

    @pl.when(j != 0)
    def _():
        o_ref[:, :] += contrib.astype(jnp.bfloat16)


def _compact_compute(e_loc2, x_loc, expert_W):
    return pl.pallas_call(
        _compact_body,
        grid=(CAP // BM_C, E_PER_SHARD),
        in_specs=[
            pl.BlockSpec((BM_C, 1), lambda c, j: (c, 0)),
            pl.BlockSpec((BM_C, D), lambda c, j: (c, 0)),
            pl.BlockSpec((1, D, H), lambda c, j: (j, 0, 0)),
        ],
        out_specs=pl.BlockSpec((BM_C, H), lambda c, j: (c, 0)),
        out_shape=jax.ShapeDtypeStruct((CAP, H), jnp.bfloat16),
    )(e_loc2, x_loc, expert_W)


def _comm_body(slab_ref, out_ref, send_sems, recv_sems, local_sem):
    my = lax.axis_index("i")
    own = pltpu.make_async_copy(slab_ref, out_ref.at[my], local_sem)
    own.start()
    copies = []
    for k in range(1, N_DEV):
        tgt = lax.rem(my + k, N_DEV)
        rdma = pltpu.make_async_remote_copy(
            src_ref=slab_ref,
            dst_ref=out_ref.at[my],
            send_sem=send_sems.at[k - 1],
            recv_sem=recv_sems.at[k - 1],
            device_id=(tgt,),
            device_id_type=pl.DeviceIdType.MESH,
        )
        rdma.start()
        copies.append(rdma)
    own.wait()
    for r in copies:
        r.wait_send()
    for r in copies:
        r.wait_recv()


def _all_gather_slabs(slab):
    return pl.pallas_call(
        _comm_body,
        out_shape=jax.ShapeDtypeStruct((N_DEV, CAP, H), jnp.bfloat16),
        in_specs=[pl.BlockSpec(memory_space=pltpu.MemorySpace.VMEM)],
        out_specs=pl.BlockSpec(memory_space=pltpu.MemorySpace.HBM),
        scratch_shapes=[
            pltpu.SemaphoreType.DMA((N_DEV - 1,)),
            pltpu.SemaphoreType.DMA((N_DEV - 1,)),
            pltpu.SemaphoreType.DMA,
        ],
    )(slab)


def _scatter_body(col_ref, slabs_ref, o_ref):
    k = pl.program_id(1)
    c = k * BK_S + lax.broadcasted_iota(jnp.int32, (1, BK_S), 1)
    s = (col_ref[:, :] == c).astype(jnp.bfloat16)
    contrib = jnp.dot(s, slabs_ref[:, :], preferred_element_type=jnp.float32)

    @pl.when(k == 0)
    def _():
        o_ref[:, :] = contrib

    @pl.when(k != 0)
    def _():
        o_ref[:, :] += contrib


def _scatter_out(col2, slabs_flat):
    return pl.pallas_call(
        _scatter_body,
        grid=(N_TOK // BM_S, (N_DEV * CAP) // BK_S),
        in_specs=[
            pl.BlockSpec((BM_S, 1), lambda m, k: (m, 0)),
            pl.BlockSpec((BK_S, H), lambda m, k: (k, 0)),
        ],
        out_specs=pl.BlockSpec((BM_S, H), lambda m, k: (m, 0)),
        out_shape=jax.ShapeDtypeStruct((N_TOK, H), jnp.float32),
    )(col2, slabs_flat)


def kernel(x, router_W, route_idx, expert_W):
    del router_W
    e = route_idx[:, 0].astype(jnp.int32)
    my = lax.axis_index("i")
    owner = e // E_PER_SHARD

    t_ids = jnp.arange(N_TOK, dtype=jnp.int32)
    not_mine = (owner != my).astype(jnp.int32)
    key = (not_mine << 17) | (t_ids << 5) | e
    skey = jnp.sort(key)[:CAP]
    my_rows = (skey >> 5) & 0xFFF
    e_loc = (skey & 0x1F) - my * E_PER_SHARD

    oh = (owner[:, None] == jnp.arange(N_DEV, dtype=jnp.int32)[None, :])
    ohi = oh.astype(jnp.int32)
    rank = jnp.sum(ohi * (jnp.cumsum(ohi, axis=0) - 1), axis=1)
    col = owner * CAP + rank

    x_loc = _gather_x(my_rows.reshape(CAP, 1), x)
    slab = _compact_compute(e_loc.reshape(CAP, 1), x_loc, expert_W)
    slabs = _all_gather_slabs(slab)
    return _scatter_out(
        col.reshape(N_TOK, 1), slabs.reshape(N_DEV * CAP, H)
    )


# device time: 343620 ns/iter; 4.1492x vs baseline; 1.0666x over previous
import jax
import jax.numpy as jnp
from jax import lax
from jax.experimental import pallas as pl
from jax.experimental.pallas import tpu as pltpu

N_DEV = 4
E_PER_SHARD = 4
N_TOK = 4096
D = 1024
H = 2048
CAP = 1280

BK_G = 1024
BM_C = 640
BM_S = 1024
BK_S = 640


def _gather_body(rows_ref, x_ref, o_ref):
    k = pl.program_id(0)
    t = k * BK_G + lax.broadcasted_iota(jnp.int32, (1, BK_G), 1)
    p = (rows_ref[:, :] == t).astype(jnp.bfloat16)
    xb = x_ref[:, :].astype(jnp.bfloat16)
    contrib = jnp.dot(p, xb, preferred_element_type=jnp.float32)

    @pl.when(k == 0)
    def _():
        o_ref[:, :] = contrib.astype(jnp.bfloat16)

    @pl.when(k != 0)
    def _():
        o_ref[:, :] += contrib.astype(jnp.bfloat16)


def _gather_x(rows2, x):
    return pl.pallas_call(
        _gather_body,
        grid=(N_TOK // BK_G,),
        in_specs=[
            pl.BlockSpec((CAP, 1), lambda k: (0, 0)),
            pl.BlockSpec((BK_G, D), lambda k: (k, 0)),
        ],
        out_specs=pl.BlockSpec((CAP, D), lambda k: (0, 0)),
        out_shape=jax.ShapeDtypeStruct((CAP, D), jnp.bfloat16),
    )(rows2, x)


def _compact_body(e_ref, x_ref, w_ref, o_ref):
    j = pl.program_id(1)
    m = e_ref[:, :] == j
    xm = jnp.where(m, x_ref[:, :], jnp.bfloat16(0.0))
    contrib = jnp.dot(xm, w_ref[0], preferred_element_type=jnp.float32)

    @pl.when(j == 0)
    def _():
        o_ref[:, :] = contrib.astype(jnp.bfloat16)

    @pl.when(j != 0)
    def _():
        o_ref[:, :] += contrib.astype(jnp.bfloat16)


def _compact_compute(e_loc2, x_loc, expert_W):
    return pl.pallas_call(
        _compact_body,
        grid=(CAP // BM_C, E_PER_SHARD),
        in_specs=[
            pl.BlockSpec((BM_C, 1), lambda c, j: (c, 0)),
            pl.BlockSpec((BM_C, D), lambda c, j: (c, 0)),
            pl.BlockSpec((1, D, H), lambda c, j: (j, 0, 0)),
        ],
        out_specs=pl.BlockSpec((BM_C, H), lambda c, j: (c, 0)),
        out_shape=jax.ShapeDtypeStruct((CAP, H), jnp.bfloat16),
    )(e_loc2, x_loc, expert_W)


def _comm_body(slab_ref, out_ref, send_sems, recv_sems, local_sem):
    my = lax.axis_index("i")
    own = pltpu.make_async_copy(slab_ref, out_ref.at[my], local_sem)
    own.start()
    copies = []
    for k in range(1, N_DEV):
        tgt = lax.rem(my + k, N_DEV)
        rdma = pltpu.make_async_remote_copy(
            src_ref=slab_ref,
            dst_ref=out_ref.at[my],
            send_sem=send_sems.at[k - 1],
            recv_sem=recv_sems.at[k - 1],
            device_id=(tgt,),
            device_id_type=pl.DeviceIdType.MESH,
        )
        rdma.start()
        copies.append(rdma)
    own.wait()
    for r in copies:
        r.wait_send()
    for r in copies:
        r.wait_recv()


def _all_gather_slabs(slab):
    return pl.pallas_call(
        _comm_body,
        out_shape=jax.ShapeDtypeStruct((N_DEV, CAP, H), jnp.bfloat16),
        in_specs=[pl.BlockSpec(memory_space=pltpu.MemorySpace.VMEM)],
        out_specs=pl.BlockSpec(memory_space=pltpu.MemorySpace.HBM),
        scratch_shapes=[
            pltpu.SemaphoreType.DMA((N_DEV - 1,)),
            pltpu.SemaphoreType.DMA((N_DEV - 1,)),
            pltpu.SemaphoreType.DMA,
        ],
    )(slab)


def _scatter_body(col_ref, slabs_ref, o_ref):
    k = pl.program_id(1)
    c = k * BK_S + lax.broadcasted_iota(jnp.int32, (1, BK_S), 1)
    s = (col_ref[:, :] == c).astype(jnp.bfloat16)
    contrib = jnp.dot(s, slabs_ref[:, :], preferred_element_type=jnp.float32)

    @pl.when(k == 0)
    def _():
        o_ref[:, :] = contrib

    @pl.when(k != 0)
    def _():
        o_ref[:, :] += contrib


def _scatter_out(col2, slabs_flat):
    return pl.pallas_call(
        _scatter_body,
        grid=(N_TOK // BM_S, (N_DEV * CAP) // BK_S),
        in_specs=[
            pl.BlockSpec((BM_S, 1), lambda m, k: (m, 0)),
            pl.BlockSpec((BK_S, H), lambda m, k: (k, 0)),
        ],
        out_specs=pl.BlockSpec((BM_S, H), lambda m, k: (m, 0)),
        out_shape=jax.ShapeDtypeStruct((N_TOK, H), jnp.float32),
    )(col2, slabs_flat)


def kernel(x, router_W, route_idx, expert_W):
    del router_W
    e = route_idx[:, 0].astype(jnp.int32)
    my = lax.axis_index("i")
    owner = e // E_PER_SHARD

    t_ids = jnp.arange(N_TOK, dtype=jnp.int32)
    not_mine = (owner != my).astype(jnp.int32)
    key = (not_mine << 17) | (t_ids << 5) | e
    skey = jnp.sort(key)[:CAP]
    my_rows = (skey >> 5) & 0xFFF
    e_loc = (skey & 0x1F) - my * E_PER_SHARD

    oh = (owner[:, None] == jnp.arange(N_DEV, dtype=jnp.int32)[None, :])
    ohi = oh.astype(jnp.int32)
    rank = jnp.sum(ohi * (jnp.cumsum(ohi, axis=0) - 1), axis=1)
    col = owner * CAP + rank

    x_loc = _gather_x(my_rows.reshape(CAP, 1), x)
    slab = _compact_compute(
        e_loc.reshape(CAP, 1), x_loc, expert_W.astype(jnp.bfloat16)
    )
    slabs = _all_gather_slabs(slab)
    return _scatter_out(
        col.reshape(N_TOK, 1), slabs.reshape(N_DEV * CAP, H)
    )
